# baseline (speedup 1.0000x reference)
import functools

import jax
import jax.numpy as jnp
from jax import lax
from jax.experimental import pallas as pl
from jax.experimental.pallas import tpu as pltpu
from jax.experimental.pallas import tpu_sc as plsc

_B, _T, _D = 16, 2048, 1024
_L16 = 16
_CH = 8
_S = 64
_K = _T // _S
_NC, _NS = 2, 16
_NW = _NC * _NS
_NSLICE = _D // _L16
_NBUF = 3
_SPB = _S // _CH

_mesh = plsc.VectorSubcoreMesh(core_axis_name="c", subcore_axis_name="s")


@functools.partial(
    pl.kernel,
    out_type=jax.ShapeDtypeStruct((_NC, _NS, _D), jnp.float32),
    mesh=_mesh,
    scratch_types=[
        pltpu.VMEM((_NBUF, _CH, _D), jnp.float32),
        pltpu.VMEM((_B * _D,), jnp.float32),
        pltpu.VMEM((_NS, _D), jnp.float32),
        pltpu.VMEM((_D,), jnp.float32),
        pltpu.VMEM((2 * _L16,), jnp.int32),
        pltpu.VMEM_SHARED((_B, _NS, _D), jnp.float32),
        pltpu.SemaphoreType.DMA,
        pltpu.SemaphoreType.DMA,
        pltpu.SemaphoreType.DMA,
    ],
)
def _sc_edge_rowsum(payload, seq_lens, out, buf, part_v, gbuf, out_v, lens_v,
                    shared, sem0, sem1, sem2):
    sems = (sem0, sem1, sem2)
    cid = lax.axis_index("c")
    sid = lax.axis_index("s")
    gwid = cid * _NS + sid

    pltpu.sync_copy(seq_lens, lens_v.at[pl.ds(0, _B)])

    def seq_body(i, _):
        ibase = i * _D

        def zbody(jj, _):
            part_v[pl.ds(ibase + jj * _L16, _L16)] = jnp.zeros(
                (_L16,), jnp.float32)
            return 0
        lax.fori_loop(0, _NSLICE, zbody, 0)

        L = lens_v[pl.ds(i, _L16)][0]
        blo = (L // _S) * _SPB
        nblk = (L + (_CH - 1)) // _CH - blo
        lo = blo + (gwid * nblk) // _NW
        cnt = blo + ((gwid + 1) * nblk) // _NW - lo

        def issue(c, b):
            start = (lo + c) * _CH
            pltpu.async_copy(payload.at[i, pl.ds(start, _CH), :],
                             buf.at[b], sems[b])

        for b in range(_NBUF):
            @pl.when(b < cnt)
            def _():
                issue(jnp.int32(b), b)

        def ring_body(g, _):
            for b in range(_NBUF):
                c = g * _NBUF + b

                @pl.when(c < cnt)
                def _():
                    pltpu.make_async_copy(payload.at[0, pl.ds(0, _CH), :],
                                          buf.at[b], sems[b]).wait()
                    start = (lo + c) * _CH
                    nv = L - start
                    nvv = jnp.broadcast_to(nv, (_L16,))
                    fvs = [jnp.where(jnp.full((_L16,), r, jnp.int32) < nvv,
                                     1.0, 0.0).astype(jnp.float32)
                           for r in range(_CH)]
                    for jj in range(_NSLICE):
                        sl = pl.ds(ibase + jj * _L16, _L16)
                        acc = part_v[sl]
                        for r in range(_CH):
                            acc = acc + buf[b, r, pl.ds(jj * _L16,
                                                        _L16)] * fvs[r]
                        part_v[sl] = acc

                    @pl.when(c + _NBUF < cnt)
                    def _():
                        issue(c + _NBUF, b)
            return 0
        lax.fori_loop(0, (cnt + (_NBUF - 1)) // _NBUF, ring_body, 0)

        pltpu.sync_copy(part_v.at[pl.ds(ibase, _D)], shared.at[i, sid])
        return 0
    lax.fori_loop(0, _B, seq_body, 0)

    plsc.subcore_barrier()

    pltpu.sync_copy(shared.at[sid], gbuf)

    def rbody(jj, _):
        sl = pl.ds(jj * _L16, _L16)
        s = jnp.zeros((_L16,), jnp.float32)
        for w in range(_NS):
            s = s + gbuf[w, sl]
        out_v[sl] = s
        return 0
    lax.fori_loop(0, _NSLICE, rbody, 0)

    pltpu.sync_copy(out_v, out.at[cid, sid])


def _tc_dense_body(lens_ref, blk_ref, out_ref):
    i = pl.program_id(0)
    k = pl.program_id(1)

    @pl.when(k == 0)
    def _():
        out_ref[...] = jnp.zeros_like(out_ref)

    L = lens_ref[i]

    @pl.when((k + 1) * _S <= L)
    def _():
        out_ref[...] += jnp.sum(blk_ref[0], axis=0, keepdims=True)[None]


def _tc_epilogue(part_ref, dense_ref, w_ref, b_ref, lens_ref, out_ref):
    wcol = jnp.sum(w_ref[...], axis=0, keepdims=True)
    rs = part_ref[0] + part_ref[1] + dense_ref[...]
    s = jnp.sum(rs * wcol, axis=1)
    lens_f = lens_ref[...].reshape(_B).astype(jnp.float32)
    bmean = jnp.sum(b_ref[...]) * (1.0 / _D)
    means = s / (lens_f * float(_D)) + bmean
    out_ref[...] = means.reshape(1, _B)


def kernel(payload, seq_lens, W, b):
    edge = _sc_edge_rowsum(payload, seq_lens)
    dense = pl.pallas_call(
        _tc_dense_body,
        grid_spec=pltpu.PrefetchScalarGridSpec(
            num_scalar_prefetch=1,
            grid=(_B, _K),
            in_specs=[
                pl.BlockSpec((1, _S, _D), lambda i, k, s: (i, k, 0)),
            ],
            out_specs=pl.BlockSpec((1, 1, _D), lambda i, k, s: (i, 0, 0)),
        ),
        out_shape=jax.ShapeDtypeStruct((_B, 1, _D), jnp.float32),
    )(seq_lens, payload).reshape(_B, _D)
    means2d = pl.pallas_call(
        _tc_epilogue,
        out_shape=jax.ShapeDtypeStruct((1, _B), jnp.float32),
    )(edge, dense, W, b.reshape(1, _D), seq_lens.reshape(1, _B))
    return means2d.reshape(_B)


# speedup vs baseline: 3.8693x; 3.8693x over previous
import functools

import jax
import jax.numpy as jnp
from jax import lax
from jax.experimental import pallas as pl
from jax.experimental.pallas import tpu as pltpu
from jax.experimental.pallas import tpu_sc as plsc

_B, _T, _D = 16, 2048, 1024
_L16 = 16
_CH = 8
_S = 512
_SUB = 64
_G = _T // _S * _B
_NSUB = _S // _SUB
_NC, _NS = 2, 16
_NSLICE = _D // _L16
_NBUF = 3

_mesh = plsc.VectorSubcoreMesh(core_axis_name="c", subcore_axis_name="s")


@functools.partial(
    pl.kernel,
    out_type=jax.ShapeDtypeStruct((_NC, _NS, _D), jnp.float32),
    mesh=_mesh,
    scratch_types=[
        pltpu.VMEM((_NBUF, _CH, _D), jnp.float32),
        pltpu.VMEM((_D,), jnp.float32),
        pltpu.VMEM((2 * _L16,), jnp.int32),
        pltpu.SemaphoreType.DMA,
        pltpu.SemaphoreType.DMA,
        pltpu.SemaphoreType.DMA,
    ],
)
def _sc_edge_rowsum(payload, seq_lens, out, buf, out_v, lens_v,
                    sem0, sem1, sem2):
    sems = (sem0, sem1, sem2)
    cid = lax.axis_index("c")
    sid = lax.axis_index("s")

    pltpu.sync_copy(seq_lens, lens_v.at[pl.ds(0, _B)])

    def zbody(jj, _):
        out_v[pl.ds(jj * _L16, _L16)] = jnp.zeros((_L16,), jnp.float32)
        return 0
    lax.fori_loop(0, _NSLICE, zbody, 0)

    L = lens_v[pl.ds(sid, _L16)][0]
    blo = (L // _SUB) * (_SUB // _CH)
    nblk = (L + (_CH - 1)) // _CH - blo
    cnt0 = (nblk + 1) // 2
    lo = blo + jnp.where(cid == 0, 0, cnt0)
    cnt = jnp.where(cid == 0, cnt0, nblk - cnt0)

    def issue(c, b):
        start = (lo + c) * _CH
        pltpu.async_copy(payload.at[sid, pl.ds(start, _CH), :],
                         buf.at[b], sems[b])

    for b in range(_NBUF):
        @pl.when(b < cnt)
        def _():
            issue(jnp.int32(b), b)

    def ring_body(g, _):
        for b in range(_NBUF):
            c = g * _NBUF + b

            @pl.when(c < cnt)
            def _():
                pltpu.make_async_copy(payload.at[0, pl.ds(0, _CH), :],
                                      buf.at[b], sems[b]).wait()
                start = (lo + c) * _CH
                nv = L - start
                nvv = jnp.broadcast_to(nv, (_L16,))
                fvs = [jnp.where(jnp.full((_L16,), r, jnp.int32) < nvv,
                                 1.0, 0.0).astype(jnp.float32)
                       for r in range(_CH)]
                for jj in range(_NSLICE):
                    sl = pl.ds(jj * _L16, _L16)
                    acc = out_v[sl]
                    for r in range(_CH):
                        acc = acc + buf[b, r, pl.ds(jj * _L16,
                                                    _L16)] * fvs[r]
                    out_v[sl] = acc

                @pl.when(c + _NBUF < cnt)
                def _():
                    issue(c + _NBUF, b)
        return 0
    lax.fori_loop(0, (cnt + (_NBUF - 1)) // _NBUF, ring_body, 0)

    pltpu.sync_copy(out_v, out.at[cid, sid])


def _tc_dense_body(meta_ref, lens_ref, blk_ref, out_ref):
    g = pl.program_id(0)
    k = meta_ref[g, 1]
    L = lens_ref[meta_ref[g, 0]]
    sval = meta_ref[g, 3]

    @pl.when(meta_ref[g, 2] == 1)
    def _():
        out_ref[...] = jnp.zeros_like(out_ref)

    acc = jnp.zeros((1, _D), jnp.float32)
    for j in range(_NSUB):
        cond = jnp.logical_and(k * _S + (j + 1) * _SUB <= L, sval == 1)
        wj = cond.astype(jnp.float32)
        acc += jnp.sum(blk_ref[0, j * _SUB:(j + 1) * _SUB],
                       axis=0, keepdims=True) * wj
    out_ref[...] += acc[None]


def _tc_epilogue(part_ref, dense_ref, w_ref, b_ref, lens_ref, out_ref):
    wcol = jnp.sum(w_ref[...], axis=0, keepdims=True)
    rs = part_ref[0] + part_ref[1] + dense_ref[:, 0, :]
    s = jnp.sum(rs * wcol, axis=1)
    lens_f = lens_ref[...].reshape(_B).astype(jnp.float32)
    bmean = jnp.sum(b_ref[...]) * (1.0 / _D)
    means = s / (lens_f * float(_D)) + bmean
    out_ref[...] = means.reshape(1, _B)


def _pair_list(seq_lens):
    full = (seq_lens // _SUB) * _SUB
    nb = jnp.maximum(1, -(-full // _S)).astype(jnp.int32)
    csum = jnp.cumsum(nb)
    g = jnp.arange(_G, dtype=jnp.int32)
    valid = g < csum[-1]
    i = jnp.minimum(jnp.searchsorted(csum, g, side="right"),
                    _B - 1).astype(jnp.int32)
    k = g - (csum[i] - nb[i])
    i_pad = jnp.full((_G,), _B - 1, jnp.int32)
    k_pad = jnp.full((_G,), nb[_B - 1] - 1, jnp.int32)
    i = jnp.where(valid, i, i_pad)
    k = jnp.where(valid, k, k_pad)
    first = jnp.logical_and(k == 0, valid).astype(jnp.int32)
    return jnp.stack([i, k, first, valid.astype(jnp.int32)], axis=1)


def kernel(payload, seq_lens, W, b):
    edge = _sc_edge_rowsum(payload, seq_lens)
    meta = _pair_list(seq_lens)
    dense = pl.pallas_call(
        _tc_dense_body,
        grid_spec=pltpu.PrefetchScalarGridSpec(
            num_scalar_prefetch=2,
            grid=(_G,),
            in_specs=[
                pl.BlockSpec((1, _S, _D),
                             lambda g, m, s: (m[g, 0], m[g, 1], 0)),
            ],
            out_specs=pl.BlockSpec((1, 1, _D),
                                   lambda g, m, s: (m[g, 0], 0, 0)),
        ),
        out_shape=jax.ShapeDtypeStruct((_B, 1, _D), jnp.float32),
    )(meta, seq_lens, payload)
    means2d = pl.pallas_call(
        _tc_epilogue,
        out_shape=jax.ShapeDtypeStruct((1, _B), jnp.float32),
    )(edge, dense, W, b.reshape(1, _D), seq_lens.reshape(1, _B))
    return means2d.reshape(_B)
